# baseline (speedup 1.0000x reference)
import functools

import jax
import jax.numpy as jnp
from jax import lax
from jax.experimental import pallas as pl
from jax.experimental.pallas import tpu as pltpu

_BB = 2


def _mask_kernel(lo_ref, hi_ref, x_ref, o_ref, *, bb, F, T):
    i = pl.program_id(0)
    rows = lax.broadcasted_iota(jnp.int32, (F, T), 0)
    for j in range(bb):
        lo = lo_ref[i * bb + j]
        hi = hi_ref[i * bb + j]
        band = (rows >= lo) & (rows < hi)
        o_ref[j] = jnp.where(band, jnp.float32(0.0), x_ref[j])


def kernel(x):
    mask_ratio = 16
    xs = jnp.squeeze(x, axis=1)
    B, F, T = xs.shape
    max_mask = F // mask_ratio
    k = jax.random.key(42)
    k1, k2 = jax.random.split(k)
    if max_mask == 1:
        f_widths = jnp.ones((B,), dtype=jnp.int32)
    else:
        f_widths = jax.random.randint(k1, (B,), 1, max_mask).astype(jnp.int32)
    u = jax.random.uniform(k2, (B,))
    f_low = jnp.floor(u * (F - f_widths).astype(jnp.float32)).astype(jnp.int32)
    f_hi = f_low + f_widths

    bb = _BB
    grid = (B // bb,)
    out = pl.pallas_call(
        functools.partial(_mask_kernel, bb=bb, F=F, T=T),
        grid_spec=pltpu.PrefetchScalarGridSpec(
            num_scalar_prefetch=2,
            grid=grid,
            in_specs=[
                pl.BlockSpec((bb, F, T), lambda i, lo, hi: (i, 0, 0)),
            ],
            out_specs=pl.BlockSpec((bb, F, T), lambda i, lo, hi: (i, 0, 0)),
        ),
        out_shape=jax.ShapeDtypeStruct((B, F, T), jnp.float32),
    )(f_low, f_hi, xs)
    return out[:, None, :, :]


# speedup vs baseline: 1.3754x; 1.3754x over previous
import functools

import jax
import jax.numpy as jnp
from jax import lax
from jax.experimental import pallas as pl
from jax.experimental.pallas import tpu as pltpu

_BB = 16


def _mask_kernel(lo_ref, hi_ref, x_ref, o_ref, *, bb, F, T):
    i = pl.program_id(0)
    rows = lax.broadcasted_iota(jnp.int32, (F, T), 0)
    for j in range(bb):
        lo = lo_ref[i * bb + j]
        hi = hi_ref[i * bb + j]
        band = (rows >= lo) & (rows < hi)
        o_ref[j] = jnp.where(band, jnp.float32(0.0), x_ref[j])


def kernel(x):
    mask_ratio = 16
    xs = jnp.squeeze(x, axis=1)
    B, F, T = xs.shape
    max_mask = F // mask_ratio
    k = jax.random.key(42)
    k1, k2 = jax.random.split(k)
    if max_mask == 1:
        f_widths = jnp.ones((B,), dtype=jnp.int32)
    else:
        f_widths = jax.random.randint(k1, (B,), 1, max_mask).astype(jnp.int32)
    u = jax.random.uniform(k2, (B,))
    f_low = jnp.floor(u * (F - f_widths).astype(jnp.float32)).astype(jnp.int32)
    f_hi = f_low + f_widths

    bb = _BB
    grid = (B // bb,)
    out = pl.pallas_call(
        functools.partial(_mask_kernel, bb=bb, F=F, T=T),
        grid_spec=pltpu.PrefetchScalarGridSpec(
            num_scalar_prefetch=2,
            grid=grid,
            in_specs=[
                pl.BlockSpec((bb, F, T), lambda i, lo, hi: (i, 0, 0)),
            ],
            out_specs=pl.BlockSpec((bb, F, T), lambda i, lo, hi: (i, 0, 0)),
        ),
        out_shape=jax.ShapeDtypeStruct((B, F, T), jnp.float32),
    )(f_low, f_hi, xs)
    return out[:, None, :, :]
